# Initial kernel scaffold; baseline (speedup 1.0000x reference)
import jax
import jax.numpy as jnp
from jax.experimental import pallas as pl


def kernel(x, edge_index, W1, as1, ad1, b1, bn1w, bn1b, rW1, rb1, W2, as2, ad2, b2, bn2w, bn2b, rW2, rb2, W3, as3, ad3, b3):
    raise NotImplementedError("write your pallas kernel here")


# speedup vs baseline: 16.0007x; 16.0007x over previous
import functools

import jax
import jax.numpy as jnp
from jax import lax
from jax.experimental import pallas as pl
from jax.experimental.pallas import tpu as pltpu
from jax.experimental.pallas import tpu_sc as plsc

N = 10000
E = 320000
F = 128
H8 = 8
NC = 2
NS = 16
NW = NC * NS
EW = E // NW
B = 80
NCH = EW // B
ROWS_T = 624
ROWS_REM = N - ROWS_T * NS

_f32 = jnp.float32
_i32 = jnp.int32


def _pre_body(x_ref, w_ref, asrc_ref, adst_ref, rx_ref, rw_ref, rb_ref,
              htab_ref, asp_ref, adp_ref, m_ref, resid_ref):
    h = jnp.dot(x_ref[...], w_ref[...], preferred_element_type=_f32)
    a_s = jnp.dot(h, asrc_ref[...], preferred_element_type=_f32)
    a_d = jnp.dot(h, adst_ref[...], preferred_element_type=_f32)
    zpad = jnp.zeros((h.shape[0], 8), _f32)
    htab_ref[...] = h
    asp_ref[...] = jnp.concatenate([a_s, zpad], axis=1)
    adp_ref[...] = jnp.concatenate([a_d, zpad], axis=1)
    mm = jnp.max(a_s, axis=0) + jnp.max(a_d, axis=0)
    m = jnp.where(mm > 0, mm, mm * _f32(0.2))
    m_ref[...] = jnp.broadcast_to(m[:, None], (8, 16))
    resid_ref[...] = (jnp.dot(rx_ref[...], rw_ref[...],
                              preferred_element_type=_f32) + rb_ref[...])


_pre_call = pl.pallas_call(
    _pre_body,
    out_shape=(
        jax.ShapeDtypeStruct((N, F), _f32),
        jax.ShapeDtypeStruct((N, 16), _f32),
        jax.ShapeDtypeStruct((N, 16), _f32),
        jax.ShapeDtypeStruct((8, 16), _f32),
        jax.ShapeDtypeStruct((N, F), _f32),
    ),
)


def _pre3_body(x_ref, w_ref, asrc_ref, adst_ref,
               htab_ref, asp_ref, adp_ref, m_ref):
    h = jnp.dot(x_ref[...], w_ref[...], preferred_element_type=_f32)
    a_s = jnp.dot(h, asrc_ref[...], preferred_element_type=_f32)
    a_d = jnp.dot(h, adst_ref[...], preferred_element_type=_f32)
    zpad = jnp.zeros((h.shape[0], 8), _f32)
    htab_ref[...] = h
    asp_ref[...] = jnp.concatenate([a_s, zpad], axis=1)
    adp_ref[...] = jnp.concatenate([a_d, zpad], axis=1)
    mm = jnp.max(a_s, axis=0) + jnp.max(a_d, axis=0)
    m = jnp.where(mm > 0, mm, mm * _f32(0.2))
    m_ref[...] = jnp.broadcast_to(m[:, None], (8, 16))


_pre3_call = pl.pallas_call(
    _pre3_body,
    out_shape=(
        jax.ShapeDtypeStruct((N, F), _f32),
        jax.ShapeDtypeStruct((N, 16), _f32),
        jax.ShapeDtypeStruct((N, 16), _f32),
        jax.ShapeDtypeStruct((8, 16), _f32),
    ),
)


RB = 1000
NG = N // RB


def _gat_block(accp_ref, denp_ref, b_ref):
    acc = accp_ref[0] + accp_ref[1]
    den = denp_ref[0] + denp_ref[1]
    denx = jnp.broadcast_to(den[:, :, None], (RB, 8, 16)).reshape(RB, F)
    return acc / (denx + _f32(1e-16)) + b_ref[...]


def _stat_body(accp_ref, denp_ref, b_ref, sum_ref, sq_ref):
    g = pl.program_id(0)
    gat = _gat_block(accp_ref, denp_ref, b_ref)
    s = jnp.sum(gat, axis=0, keepdims=True)
    q = jnp.sum(gat * gat, axis=0, keepdims=True)

    @pl.when(g == 0)
    def _():
        sum_ref[...] = s
        sq_ref[...] = q

    @pl.when(g > 0)
    def _():
        sum_ref[...] += s
        sq_ref[...] += q


def _apply_body(accp_ref, denp_ref, b_ref, bnw_ref, bnb_ref, resid_ref,
                sum_ref, sq_ref, y_ref, xn_ref):
    gat = _gat_block(accp_ref, denp_ref, b_ref)
    mean = sum_ref[...] * _f32(1.0 / N)
    var = sq_ref[...] * _f32(1.0 / N) - mean * mean
    bn = (gat - mean) / jnp.sqrt(var + _f32(1e-5)) * bnw_ref[...] + bnb_ref[...]
    y = bn + resid_ref[...]
    y_ref[...] = y
    xn_ref[...] = jnp.maximum(y, _f32(0.0))


_accp_spec = pl.BlockSpec((NC, RB, F), lambda g: (0, g, 0))
_denp_spec = pl.BlockSpec((NC, RB, 8), lambda g: (0, g, 0))
_row_spec = pl.BlockSpec((1, F), lambda g: (0, 0))
_blk_spec = pl.BlockSpec((RB, F), lambda g: (g, 0))

_stat_call = pl.pallas_call(
    _stat_body,
    grid=(NG,),
    in_specs=[_accp_spec, _denp_spec, _row_spec],
    out_specs=(_row_spec, _row_spec),
    out_shape=(
        jax.ShapeDtypeStruct((1, F), _f32),
        jax.ShapeDtypeStruct((1, F), _f32),
    ),
)

_apply_call = pl.pallas_call(
    _apply_body,
    grid=(NG,),
    in_specs=[_accp_spec, _denp_spec, _row_spec, _row_spec, _row_spec,
              _blk_spec, _row_spec, _row_spec],
    out_specs=(_blk_spec, _blk_spec),
    out_shape=(
        jax.ShapeDtypeStruct((N, F), _f32),
        jax.ShapeDtypeStruct((N, F), _f32),
    ),
)


def _post_call(accp, denp, b, bnw, bnb, resid):
    ssum, ssq = _stat_call(accp, denp, b)
    return _apply_call(accp, denp, b, bnw, bnb, resid, ssum, ssq)


def _post3_body(accp_ref, denp_ref, b_ref, out_ref):
    acc = accp_ref[0] + accp_ref[1]
    den = denp_ref[0][:, 0:1] + denp_ref[1][:, 0:1]
    gat = acc / (den + _f32(1e-16)) + b_ref[...]
    mx = jnp.max(gat, axis=1, keepdims=True)
    s = gat - mx
    out_ref[...] = s - jnp.log(jnp.sum(jnp.exp(s), axis=1, keepdims=True))


_post3_call = pl.pallas_call(
    _post3_body,
    out_shape=jax.ShapeDtypeStruct((N, F), _f32),
)


def _make_sc(num_heads):
    mesh = plsc.VectorSubcoreMesh(core_axis_name="c", subcore_axis_name="s",
                                  num_cores=NC, num_subcores=NS)
    heads = list(range(num_heads))

    @functools.partial(
        pl.kernel,
        compiler_params=pltpu.CompilerParams(use_tc_tiling_on_sc=False,
                                             needs_layout_passes=False),
        out_type=(
            jax.ShapeDtypeStruct((NC, N, F), _f32),
            jax.ShapeDtypeStruct((NC, N, 8), _f32),
        ),
        mesh=mesh,
        scratch_types=[
            pltpu.VMEM((NCH, B), _i32),
            pltpu.VMEM((NCH, B), _i32),
            pltpu.VMEM((B, F), _f32),
            pltpu.VMEM((B, 16), _f32),
            pltpu.VMEM((B, 16), _f32),
            pltpu.VMEM((B, F), _f32),
            pltpu.VMEM((B, 8), _f32),
            pltpu.VMEM((8, 16), _f32),
            pltpu.VMEM_SHARED((N, F), _f32),
            pltpu.VMEM_SHARED((N, 8), _f32),
            pltpu.SemaphoreType.DMA,
            pltpu.SemaphoreType.DMA,
            pltpu.SemaphoreType.DMA,
        ],
    )
    def sc_kernel(htab, asp, adp, m, src3, dst3, zacc, zden, accp_o, denp_o,
                  srcv, dstv, hbuf, sbuf, abuf, msgbuf, eabuf, mv,
                  acc_s, den_s, sem1, sem2, sem3):
        cid = lax.axis_index("c")
        sid = lax.axis_index("s")
        wid = cid * NS + sid
        r0 = sid * ROWS_T

        pltpu.sync_copy(zacc.at[pl.ds(r0, ROWS_T)], acc_s.at[pl.ds(r0, ROWS_T)])
        pltpu.sync_copy(zden.at[pl.ds(r0, ROWS_T)], den_s.at[pl.ds(r0, ROWS_T)])

        @pl.when(sid == NS - 1)
        def _():
            base = NS * ROWS_T
            pltpu.sync_copy(zacc.at[pl.ds(base, ROWS_REM)],
                            acc_s.at[pl.ds(base, ROWS_REM)])
            pltpu.sync_copy(zden.at[pl.ds(base, ROWS_REM)],
                            den_s.at[pl.ds(base, ROWS_REM)])

        pltpu.sync_copy(src3.at[wid], srcv)
        pltpu.sync_copy(dst3.at[wid], dstv)
        pltpu.sync_copy(m, mv)
        pltpu.sync_copy(zden.at[pl.ds(0, B)], eabuf)
        plsc.subcore_barrier()

        iota = lax.iota(_i32, 16)
        mb = [mv[k] for k in heads]

        def chunk_body(ci, carry):
            cp1 = pltpu.async_copy(htab.at[srcv.at[ci]], hbuf, sem1)
            cp2 = pltpu.async_copy(asp.at[srcv.at[ci]], sbuf, sem2)
            cp3 = pltpu.async_copy(adp.at[dstv.at[ci]], abuf, sem3)
            cp1.wait()
            cp2.wait()
            cp3.wait()

            def group_body(g, gcarry):
                eidx = g * 16 + iota
                ea = []
                for k in heads:
                    kvec = jnp.full((16,), k, _i32)
                    s_k = plsc.load_gather(sbuf, [eidx, kvec])
                    d_k = plsc.load_gather(abuf, [eidx, kvec])
                    z = s_k + d_k
                    al = jnp.where(z > 0, z, z * _f32(0.2))
                    ea_k = jnp.exp(al - mb[k])
                    plsc.store_scatter(eabuf, [eidx, kvec], ea_k)
                    ea.append(ea_k)
                for c in range(F):
                    kk = (c // 16) if num_heads == H8 else 0
                    cvec = jnp.full((16,), c, _i32)
                    hcol = plsc.load_gather(hbuf, [eidx, cvec])
                    plsc.store_scatter(msgbuf, [eidx, cvec], hcol * ea[kk])
                return gcarry

            lax.fori_loop(0, B // 16, group_body, 0)
            pltpu.sync_copy(msgbuf, acc_s.at[dstv.at[ci]], add=True)
            pltpu.sync_copy(eabuf, den_s.at[dstv.at[ci]], add=True)
            return carry

        lax.fori_loop(0, NCH, chunk_body, 0)
        plsc.subcore_barrier()
        pltpu.sync_copy(acc_s.at[pl.ds(r0, ROWS_T)],
                        accp_o.at[cid, pl.ds(r0, ROWS_T)])
        pltpu.sync_copy(den_s.at[pl.ds(r0, ROWS_T)],
                        denp_o.at[cid, pl.ds(r0, ROWS_T)])

        @pl.when(sid == NS - 1)
        def _():
            base = NS * ROWS_T
            pltpu.sync_copy(acc_s.at[pl.ds(base, ROWS_REM)],
                            accp_o.at[cid, pl.ds(base, ROWS_REM)])
            pltpu.sync_copy(den_s.at[pl.ds(base, ROWS_REM)],
                            denp_o.at[cid, pl.ds(base, ROWS_REM)])

    return sc_kernel


_sc_h8 = _make_sc(8)
_sc_h1 = _make_sc(1)


def _att_matrix(att):
    h, c = att.shape
    return (jnp.eye(h, dtype=_f32)[:, None, :] * att[:, :, None]).reshape(h * c, h)


def kernel(x, edge_index, W1, as1, ad1, b1, bn1w, bn1b, rW1, rb1,
           W2, as2, ad2, b2, bn2w, bn2b, rW2, rb2, W3, as3, ad3, b3):
    src3 = edge_index[0].reshape(NW, NCH, B)
    dst3 = edge_index[1].reshape(NW, NCH, B)
    zacc = jnp.zeros((N, F), _f32)
    zden = jnp.zeros((N, 8), _f32)

    A1s, A1d = _att_matrix(as1), _att_matrix(ad1)
    A2s, A2d = _att_matrix(as2), _att_matrix(ad2)
    A3s = jnp.pad(_att_matrix(as3), ((0, 0), (0, 7)))
    A3d = jnp.pad(_att_matrix(ad3), ((0, 0), (0, 7)))

    htab, asp, adp, m, resid = _pre_call(x, W1, A1s, A1d, x, rW1, rb1[None, :])
    accp, denp = _sc_h8(htab, asp, adp, m, src3, dst3, zacc, zden)
    y1, x2 = _post_call(accp, denp, b1[None, :], bn1w[None, :], bn1b[None, :],
                        resid)
    htab, asp, adp, m, resid = _pre_call(x2, W2, A2s, A2d, y1, rW2,
                                         rb2[None, :])
    accp, denp = _sc_h8(htab, asp, adp, m, src3, dst3, zacc, zden)
    _, x3 = _post_call(accp, denp, b2[None, :], bn2w[None, :], bn2b[None, :],
                       resid)
    htab, asp, adp, m = _pre3_call(x3, W3, A3s, A3d)
    accp, denp = _sc_h1(htab, asp, adp, m, src3, dst3, zacc, zden)
    return _post3_call(accp, denp, b3[None, :])
